# baseline (speedup 1.0000x reference)
import functools

import jax
import jax.numpy as jnp
from jax import lax
from jax.experimental import pallas as pl
from jax.experimental.pallas import tpu as pltpu
from jax.experimental.pallas import tpu_sc as plsc

BATCH = 1024
EMBED = 768
PARTITION = 96
CENTROIDS = 256
DSUB = 8
GROUP = 16
NGROUPS = PARTITION // GROUP
SEG = GROUP * CENTROIDS

NCORES = 2
NSUBCORES = 16
NWORKERS = NCORES * NSUBCORES
ROWS = BATCH * PARTITION
ROWS_PER_W = ROWS // NWORKERS


def _argmin_group(vec_ref, cbt_ref, idx_ref, w_ref):
    g = pl.program_id(0)
    w_ref[...] = jnp.zeros((GROUP * DSUB, SEG), jnp.float32)
    for q in range(GROUP):
        w_ref[q * DSUB:(q + 1) * DSUB, q * CENTROIDS:(q + 1) * CENTROIDS] = cbt_ref[q]

    w = w_ref[...]
    cnorm = jnp.sum(w * w, axis=0, keepdims=True)
    v = vec_ref[...]
    scores = jax.lax.dot_general(
        v, w, (((1,), (0,)), ((), ())),
        precision=jax.lax.Precision.HIGHEST,
        preferred_element_type=jnp.float32)
    adj = 2.0 * scores - cnorm

    cols = []
    for q in range(GROUP):
        seg = adj[:, q * CENTROIDS:(q + 1) * CENTROIDS]
        m = jnp.max(seg, axis=1, keepdims=True)
        iota = jax.lax.broadcasted_iota(jnp.int32, seg.shape, 1)
        cand = jnp.where(seg == m, iota, CENTROIDS)
        idx = jnp.min(cand, axis=1, keepdims=True)
        cols.append(idx + (g * GROUP + q) * CENTROIDS)
    idx_ref[0] = jnp.concatenate(cols, axis=1)


def _nearest_indices(vecs, cbt):
    return pl.pallas_call(
        _argmin_group,
        grid=(NGROUPS,),
        in_specs=[
            pl.BlockSpec((BATCH, GROUP * DSUB), lambda g: (0, g)),
            pl.BlockSpec((GROUP, DSUB, CENTROIDS), lambda g: (g, 0, 0)),
        ],
        out_specs=pl.BlockSpec((1, BATCH, GROUP), lambda g: (g, 0, 0)),
        out_shape=jax.ShapeDtypeStruct((NGROUPS, BATCH, GROUP), jnp.int32),
        scratch_shapes=[pltpu.VMEM((GROUP * DSUB, SEG), jnp.float32)],
    )(vecs, cbt)


ROW_PAD = 16


@functools.cache
def _sc_gather():
    @functools.partial(
        pl.kernel,
        mesh=plsc.VectorSubcoreMesh(core_axis_name="c", subcore_axis_name="s"),
        compiler_params=pltpu.CompilerParams(use_tc_tiling_on_sc=False),
        out_type=jax.ShapeDtypeStruct((ROWS, ROW_PAD), jnp.float32),
        scratch_types=[
            pltpu.VMEM((ROWS_PER_W,), jnp.int32),
            pltpu.VMEM((ROWS_PER_W, ROW_PAD), jnp.float32),
            pltpu.SemaphoreType.DMA,
        ],
    )
    def gather(table_hbm, idx_hbm, out_hbm, idx_v, rows_v, sem):
        wid = lax.axis_index("s") * NCORES + lax.axis_index("c")
        base = wid * ROWS_PER_W
        pltpu.sync_copy(idx_hbm.at[pl.ds(base, ROWS_PER_W)], idx_v)
        pltpu.async_copy(table_hbm.at[idx_v], rows_v, sem).wait()
        pltpu.sync_copy(rows_v, out_hbm.at[pl.ds(base, ROWS_PER_W)])

    return gather


@jax.jit
def kernel(vecs, codebook):
    cbt = codebook.transpose(0, 2, 1)
    flat_idx = _nearest_indices(vecs, cbt).transpose(1, 0, 2).reshape(ROWS)
    table = jnp.pad(codebook.reshape(PARTITION * CENTROIDS, DSUB),
                    ((0, 0), (0, ROW_PAD - DSUB)))
    rows = _sc_gather()(table, flat_idx)
    return rows[:, :DSUB].reshape(BATCH, EMBED)


# speedup vs baseline: 1.9554x; 1.9554x over previous
import jax
import jax.numpy as jnp
from jax.experimental import pallas as pl
from jax.experimental.pallas import tpu as pltpu

BATCH = 1024
EMBED = 768
PARTITION = 96
CENTROIDS = 256
DSUB = 8
GROUP = 16
NGROUPS = PARTITION // GROUP
SEG = GROUP * CENTROIDS


def _quant_group(vec_ref, cbt_ref, cb_ref, out_ref, w_ref, c_ref):
    w_ref[...] = jnp.zeros((GROUP * DSUB, SEG), jnp.float32)
    c_ref[...] = jnp.zeros((SEG, GROUP * DSUB), jnp.bfloat16)
    for q in range(GROUP):
        w_ref[q * DSUB:(q + 1) * DSUB, q * CENTROIDS:(q + 1) * CENTROIDS] = cbt_ref[q]
        c_ref[q * CENTROIDS:(q + 1) * CENTROIDS, q * DSUB:(q + 1) * DSUB] = (
            cb_ref[q].astype(jnp.bfloat16))

    w = w_ref[...]
    cnorm = jnp.sum(w * w, axis=0, keepdims=True)
    v = vec_ref[...]
    scores = jax.lax.dot_general(
        v, w, (((1,), (0,)), ((), ())),
        precision=jax.lax.Precision.HIGHEST,
        preferred_element_type=jnp.float32)
    adj = 2.0 * scores - cnorm

    iota = jax.lax.broadcasted_iota(
        jnp.int32, (BATCH, CENTROIDS), 1).astype(jnp.float32)
    hots = []
    for q in range(GROUP):
        seg = adj[:, q * CENTROIDS:(q + 1) * CENTROIDS]
        m = jnp.max(seg, axis=1, keepdims=True)
        cand = jnp.where(seg == m, iota, float(CENTROIDS))
        idx = jnp.min(cand, axis=1, keepdims=True)
        hots.append((iota == idx).astype(jnp.bfloat16))
    hot = jnp.concatenate(hots, axis=1)
    out_ref[...] = jax.lax.dot_general(
        hot, c_ref[...], (((1,), (0,)), ((), ())),
        preferred_element_type=jnp.float32)


@jax.jit
def kernel(vecs, codebook):
    cbt = codebook.transpose(0, 2, 1)
    return pl.pallas_call(
        _quant_group,
        grid=(NGROUPS,),
        in_specs=[
            pl.BlockSpec((BATCH, GROUP * DSUB), lambda g: (0, g)),
            pl.BlockSpec((GROUP, DSUB, CENTROIDS), lambda g: (g, 0, 0)),
            pl.BlockSpec((GROUP, CENTROIDS, DSUB), lambda g: (g, 0, 0)),
        ],
        out_specs=pl.BlockSpec((BATCH, GROUP * DSUB), lambda g: (0, g)),
        out_shape=jax.ShapeDtypeStruct((BATCH, EMBED), jnp.float32),
        scratch_shapes=[
            pltpu.VMEM((GROUP * DSUB, SEG), jnp.float32),
            pltpu.VMEM((SEG, GROUP * DSUB), jnp.bfloat16),
        ],
    )(vecs, cbt, codebook)
